# Initial kernel scaffold; baseline (speedup 1.0000x reference)
import jax
import jax.numpy as jnp
from jax.experimental import pallas as pl


def kernel(x, edge_index, W1, b1, W2, b2, W3, b3, Wc, bc):
    raise NotImplementedError("write your pallas kernel here")


# speedup vs baseline: 20.5621x; 20.5621x over previous
import functools
import math

import jax
import jax.numpy as jnp
from jax import lax
from jax.experimental import pallas as pl
from jax.experimental.pallas import tpu as pltpu
from jax.experimental.pallas import tpu_sc as plsc

NC = 2
NS = 16
NW = NC * NS
IDXW = 128
CR = 8
BN = 1024


def _mesh():
    return plsc.VectorSubcoreMesh(
        core_axis_name="c", subcore_axis_name="s",
        num_cores=NC, num_subcores=NS)


def _deg_body(dst_hbm, ones_hbm, zeros_hbm, deg_out, acc_sh, idx_v, ones_v):
    cid = lax.axis_index("c")
    sid = lax.axis_index("s")
    wid = cid * NS + sid
    npad = acc_sh.shape[0]
    slc = npad // NS
    pltpu.sync_copy(ones_hbm, ones_v)
    pltpu.sync_copy(zeros_hbm, acc_sh.at[pl.ds(sid * slc, slc)])
    plsc.subcore_barrier()
    rows_per_tile = dst_hbm.shape[0] // NW
    base = wid * rows_per_tile

    def outer(g, carry):
        pltpu.sync_copy(dst_hbm.at[pl.ds(base + g * CR, CR)], idx_v)
        for j in range(CR):
            pltpu.sync_copy(ones_v, acc_sh.at[idx_v.at[j]], add=True)
        return carry

    lax.fori_loop(0, rows_per_tile // CR, outer, 0)
    plsc.subcore_barrier()
    pltpu.sync_copy(acc_sh.at[pl.ds(sid * slc, slc)],
                    deg_out.at[cid, pl.ds(sid * slc, slc)])


def _agg_body(src_hbm, dst_hbm, zt_hbm, zeros_hbm, q_out,
              acc_sh, sidx_v, didx_v, rows_v, sem):
    cid = lax.axis_index("c")
    sid = lax.axis_index("s")
    wid = cid * NS + sid
    npad = acc_sh.shape[0]
    slc = npad // NS
    pltpu.sync_copy(zeros_hbm, acc_sh.at[pl.ds(sid * slc, slc)])
    plsc.subcore_barrier()
    rows_per_tile = src_hbm.shape[0] // NW
    base = wid * rows_per_tile

    def outer(g, carry):
        pltpu.sync_copy(src_hbm.at[pl.ds(base + g * CR, CR)], sidx_v)
        pltpu.sync_copy(dst_hbm.at[pl.ds(base + g * CR, CR)], didx_v)
        for j in range(CR):
            pltpu.async_copy(zt_hbm.at[sidx_v.at[j]], rows_v, sem).wait()
            pltpu.sync_copy(rows_v, acc_sh.at[didx_v.at[j]], add=True)
        return carry

    lax.fori_loop(0, rows_per_tile // CR, outer, 0)
    plsc.subcore_barrier()
    pltpu.sync_copy(acc_sh.at[pl.ds(sid * slc, slc)],
                    q_out.at[cid, pl.ds(sid * slc, slc)])


_SC_PARAMS = pltpu.CompilerParams(use_tc_tiling_on_sc=False)


def _make_deg(npad, nrows):
    return pl.kernel(
        _deg_body,
        out_type=jax.ShapeDtypeStruct((NC, npad), jnp.float32),
        mesh=_mesh(),
        compiler_params=_SC_PARAMS,
        scratch_types=[
            pltpu.VMEM_SHARED((npad,), jnp.float32),
            pltpu.VMEM((CR, IDXW), jnp.int32),
            pltpu.VMEM((IDXW,), jnp.float32),
        ],
    )


def _make_agg(npad, d):
    return pl.kernel(
        _agg_body,
        out_type=jax.ShapeDtypeStruct((NC, npad, d), jnp.float32),
        mesh=_mesh(),
        compiler_params=_SC_PARAMS,
        scratch_types=[
            pltpu.VMEM_SHARED((npad, d), jnp.float32),
            pltpu.VMEM((CR, IDXW), jnp.int32),
            pltpu.VMEM((CR, IDXW), jnp.int32),
            pltpu.VMEM((IDXW, d), jnp.float32),
            pltpu.SemaphoreType.DMA,
        ],
    )


def _dense0_body(p_ref, x_ref, w_ref, b_ref,
                 alpha_ref, selfw_ref, zt_ref, s_ref):
    deg = p_ref[0, :] + p_ref[1, :] + 1.0
    alpha = lax.rsqrt(deg)
    selfw = 1.0 / deg
    z = jnp.dot(x_ref[...], w_ref[...], preferred_element_type=jnp.float32)
    alpha_ref[...] = alpha
    selfw_ref[...] = selfw
    zt_ref[...] = z * alpha[:, None]
    s_ref[...] = z * selfw[:, None] + b_ref[...]


def _dense_mid_body(q0_ref, q1_ref, alpha_ref, selfw_ref, s_ref, w_ref, b_ref,
                    zt_ref, snext_ref):
    alpha = alpha_ref[...]
    h = jnp.tanh(alpha[:, None] * (q0_ref[...] + q1_ref[...]) + s_ref[...])
    z = jnp.dot(h, w_ref[...], preferred_element_type=jnp.float32)
    zt_ref[...] = z * alpha[:, None]
    snext_ref[...] = z * selfw_ref[...][:, None] + b_ref[...]


def _dense_out_body(q0_ref, q1_ref, alpha_ref, s_ref, w_ref, b_ref,
                    out_ref, emb_ref):
    emb = jnp.tanh(alpha_ref[...][:, None] * (q0_ref[...] + q1_ref[...])
                   + s_ref[...])
    emb_ref[...] = emb[:, :emb_ref.shape[1]]
    out_ref[...] = (jnp.dot(emb, w_ref[...], preferred_element_type=jnp.float32)
                    + b_ref[...])


def _row_spec(d):
    return pl.BlockSpec((BN, d), lambda i: (i, 0))


def _vec_spec():
    return pl.BlockSpec((BN,), lambda i: (i,))


def _full_spec(shape):
    return pl.BlockSpec(shape, lambda i: tuple(0 for _ in shape))


def kernel(x, edge_index, W1, b1, W2, b2, W3, b3, Wc, bc):
    n, in_dim = x.shape
    e = edge_index.shape[1]
    hid = W1.shape[1]
    emb_d = W3.shape[1]
    ncls = Wc.shape[1]
    AD = 8
    npad = (n + BN) // BN * BN
    rows_per_tile = math.ceil(e / (NW * IDXW * CR)) * CR
    epad = NW * IDXW * rows_per_tile
    grid = (npad // BN,)

    pad_idx = n + jnp.arange(epad - e, dtype=jnp.int32) % (npad - n)
    src2d = jnp.concatenate([edge_index[0], pad_idx]).reshape(epad // IDXW, IDXW)
    dst2d = jnp.concatenate([edge_index[1], pad_idx]).reshape(epad // IDXW, IDXW)
    xp = jnp.zeros((npad, in_dim), jnp.float32).at[:n].set(x)
    ones = jnp.ones((IDXW,), jnp.float32)
    zeros1 = jnp.zeros((npad // NS,), jnp.float32)
    zeros_h = jnp.zeros((npad // NS, AD), jnp.float32)

    def padw(w, b):
        wp = jnp.zeros((w.shape[0], AD), jnp.float32).at[:, :w.shape[1]].set(w)
        bp = jnp.zeros((1, AD), jnp.float32).at[:, :b.shape[0]].set(b)
        return wp, bp

    W1p, b1p = padw(W1, b1)
    W2p, b2p = padw(jnp.zeros((AD, hid), jnp.float32).at[:hid].set(W2), b2)
    W3p, b3p = padw(jnp.zeros((AD, emb_d), jnp.float32).at[:hid].set(W3), b3)
    Wcp = jnp.zeros((AD, ncls), jnp.float32).at[:emb_d].set(Wc)

    degp = _make_deg(npad, epad // IDXW)(dst2d, ones, zeros1)

    dense0 = pl.pallas_call(
        _dense0_body,
        grid=grid,
        in_specs=[pl.BlockSpec((NC, BN), lambda i: (0, i)),
                  _row_spec(in_dim), _full_spec((in_dim, AD)),
                  _full_spec((1, AD))],
        out_specs=(_vec_spec(), _vec_spec(), _row_spec(AD), _row_spec(AD)),
        out_shape=(jax.ShapeDtypeStruct((npad,), jnp.float32),
                   jax.ShapeDtypeStruct((npad,), jnp.float32),
                   jax.ShapeDtypeStruct((npad, AD), jnp.float32),
                   jax.ShapeDtypeStruct((npad, AD), jnp.float32)),
    )
    alpha, selfw, zt1, s1 = dense0(degp, xp, W1p, b1p)

    agg = _make_agg(npad, AD)
    q1 = agg(src2d, dst2d, zt1, zeros_h)

    def dense_mid(q, s, wp, bp):
        f = pl.pallas_call(
            _dense_mid_body,
            grid=grid,
            in_specs=[_row_spec(AD), _row_spec(AD), _vec_spec(),
                      _vec_spec(), _row_spec(AD),
                      _full_spec((AD, AD)), _full_spec((1, AD))],
            out_specs=(_row_spec(AD), _row_spec(AD)),
            out_shape=(jax.ShapeDtypeStruct((npad, AD), jnp.float32),
                       jax.ShapeDtypeStruct((npad, AD), jnp.float32)),
        )
        return f(q[0], q[1], alpha, selfw, s, wp, bp)

    zt2, s2 = dense_mid(q1, s1, W2p, b2p)
    q2 = agg(src2d, dst2d, zt2, zeros_h)
    zt3, s3 = dense_mid(q2, s2, W3p, b3p)
    q3 = agg(src2d, dst2d, zt3, zeros_h)

    dense_out = pl.pallas_call(
        _dense_out_body,
        grid=grid,
        in_specs=[_row_spec(AD), _row_spec(AD), _vec_spec(),
                  _row_spec(AD), _full_spec((AD, ncls)),
                  _full_spec((1, ncls))],
        out_specs=(_row_spec(ncls), _row_spec(emb_d)),
        out_shape=(jax.ShapeDtypeStruct((npad, ncls), jnp.float32),
                   jax.ShapeDtypeStruct((npad, emb_d), jnp.float32)),
    )
    out, emb = dense_out(q3[0], q3[1], alpha, s3, Wcp, bc.reshape(1, ncls))
    return out[:n], emb[:n]


# speedup vs baseline: 31.9825x; 1.5554x over previous
import functools
import math

import jax
import jax.numpy as jnp
from jax import lax
from jax.experimental import pallas as pl
from jax.experimental.pallas import tpu as pltpu
from jax.experimental.pallas import tpu_sc as plsc

NC = 2
NS = 16
NW = NC * NS
IDXW = 128
BN = 1024


def _mesh():
    return plsc.VectorSubcoreMesh(
        core_axis_name="c", subcore_axis_name="s",
        num_cores=NC, num_subcores=NS)


NBUF = 4
LOOKAHEAD = 2
CB = 40
NIB = 2

_SC_PARAMS = pltpu.CompilerParams(use_tc_tiling_on_sc=False)


def _make_deg(npad, nrows):
    rpt = nrows // NW
    nblk = rpt // CB

    def body(dst_hbm, ones_hbm, zeros_hbm, deg_out, acc_sh, didx, ones_v,
             *sems):
        ssems = sems[:NBUF]
        isems = sems[NBUF:]
        cid = lax.axis_index("c")
        sid = lax.axis_index("s")
        wid = cid * NS + sid
        slc = npad // NS
        base = wid * rpt

        def iload(g, p):
            pltpu.async_copy(dst_hbm.at[pl.ds(base + g * CB, CB)],
                             didx.at[p], isems[p])

        def iwait(p):
            pltpu.make_async_copy(dst_hbm.at[pl.ds(0, CB)], didx.at[p],
                                  isems[p]).wait()

        def scat(p, j, k):
            pltpu.async_copy(ones_v, acc_sh.at[didx.at[p, j]], ssems[k],
                             add=True)

        def swait(k):
            pltpu.make_async_copy(ones_v, acc_sh.at[didx.at[0, 0]],
                                  ssems[k]).wait()

        pltpu.sync_copy(ones_hbm, ones_v)
        pltpu.sync_copy(zeros_hbm, acc_sh.at[pl.ds(sid * slc, slc)])
        iload(0, 0)
        iwait(0)
        plsc.subcore_barrier()

        def outer(gg, carry):
            for pp in range(NIB):
                g = gg * NIB + pp
                p, pn = pp, (pp + 1) % NIB
                for j in range(CB):
                    r = g * CB + j
                    k = j % NBUF

                    @pl.when(r >= NBUF)
                    def _():
                        swait(k)

                    scat(p, j, k)
                    if j == 8:
                        @pl.when(g + 1 < nblk)
                        def _():
                            iload(g + 1, pn)
                    if j == CB - 2:
                        @pl.when(g + 1 < nblk)
                        def _():
                            iwait(pn)
            return carry

        lax.fori_loop(0, nblk // NIB, outer, 0)
        for k in range(NBUF):
            swait(k)
        plsc.subcore_barrier()
        pltpu.sync_copy(acc_sh.at[pl.ds(sid * slc, slc)],
                        deg_out.at[cid, pl.ds(sid * slc, slc)])

    return pl.kernel(
        body,
        out_type=jax.ShapeDtypeStruct((NC, npad), jnp.float32),
        mesh=_mesh(),
        compiler_params=_SC_PARAMS,
        scratch_types=[
            pltpu.VMEM_SHARED((npad,), jnp.float32),
            pltpu.VMEM((NIB, CB, IDXW), jnp.int32),
            pltpu.VMEM((IDXW,), jnp.float32),
        ] + [pltpu.SemaphoreType.DMA] * (NBUF + NIB),
    )


def _make_agg(npad, d, nrows):
    rpt = nrows // NW
    nblk = rpt // CB

    def body(src_hbm, dst_hbm, zt_hbm, zeros_hbm, q_out,
             acc_sh, sidx, didx, rows, *sems):
        gsems = sems[:NBUF]
        ssems = sems[NBUF:2 * NBUF]
        isems = sems[2 * NBUF:]
        cid = lax.axis_index("c")
        sid = lax.axis_index("s")
        wid = cid * NS + sid
        slc = npad // NS
        base = wid * rpt

        def iload(g, p):
            pltpu.async_copy(src_hbm.at[pl.ds(base + g * CB, CB)],
                             sidx.at[p], isems[p])
            pltpu.async_copy(dst_hbm.at[pl.ds(base + g * CB, CB)],
                             didx.at[p], isems[p])

        def iwait(p):
            pltpu.make_async_copy(src_hbm.at[pl.ds(0, CB)], sidx.at[p],
                                  isems[p]).wait()
            pltpu.make_async_copy(dst_hbm.at[pl.ds(0, CB)], didx.at[p],
                                  isems[p]).wait()

        def gather(p, j, k):
            pltpu.async_copy(zt_hbm.at[sidx.at[p, j]], rows.at[k], gsems[k])

        def gwait(p, j, k):
            pltpu.make_async_copy(zt_hbm.at[sidx.at[p, j]], rows.at[k],
                                  gsems[k]).wait()

        def scat(p, j, k):
            pltpu.async_copy(rows.at[k], acc_sh.at[didx.at[p, j]], ssems[k],
                             add=True)

        def swait(k):
            pltpu.make_async_copy(rows.at[k], acc_sh.at[didx.at[0, 0]],
                                  ssems[k]).wait()

        pltpu.sync_copy(zeros_hbm, acc_sh.at[pl.ds(sid * slc, slc)])
        iload(0, 0)
        iwait(0)
        for j in range(LOOKAHEAD):
            gather(0, j, j)
        plsc.subcore_barrier()

        def outer(gg, carry):
            for pp in range(NIB):
                g = gg * NIB + pp
                p, pn = pp, (pp + 1) % NIB
                for j in range(CB):
                    r = g * CB + j
                    k = j % NBUF
                    nxt = r + LOOKAHEAD
                    jn = j + LOOKAHEAD
                    kn = jn % NBUF

                    @pl.when(jnp.logical_and(nxt >= NBUF, nxt < rpt))
                    def _():
                        swait(kn)

                    if jn < CB:
                        @pl.when(nxt < rpt)
                        def _():
                            gather(p, jn, kn)
                    else:
                        if jn == CB:
                            @pl.when(g + 1 < nblk)
                            def _():
                                iwait(pn)

                        @pl.when(nxt < rpt)
                        def _():
                            gather(pn, jn - CB, kn)

                    gwait(p, j, k)
                    scat(p, j, k)
                    if j == 8:
                        @pl.when(g + 1 < nblk)
                        def _():
                            iload(g + 1, pn)
            return carry

        lax.fori_loop(0, nblk // NIB, outer, 0)
        for k in range(NBUF):
            swait(k)
        plsc.subcore_barrier()
        pltpu.sync_copy(acc_sh.at[pl.ds(sid * slc, slc)],
                        q_out.at[cid, pl.ds(sid * slc, slc)])

    return pl.kernel(
        body,
        out_type=jax.ShapeDtypeStruct((NC, npad, d), jnp.float32),
        mesh=_mesh(),
        compiler_params=_SC_PARAMS,
        scratch_types=[
            pltpu.VMEM_SHARED((npad, d), jnp.float32),
            pltpu.VMEM((NIB, CB, IDXW), jnp.int32),
            pltpu.VMEM((NIB, CB, IDXW), jnp.int32),
            pltpu.VMEM((NBUF, IDXW, d), jnp.float32),
        ] + [pltpu.SemaphoreType.DMA] * (2 * NBUF + NIB),
    )


def _dense0_body(p_ref, x_ref, w_ref, b_ref,
                 alpha_ref, selfw_ref, zt_ref, s_ref):
    deg = p_ref[0, :] + p_ref[1, :] + 1.0
    alpha = lax.rsqrt(deg)
    selfw = 1.0 / deg
    z = jnp.dot(x_ref[...], w_ref[...], preferred_element_type=jnp.float32)
    alpha_ref[...] = alpha
    selfw_ref[...] = selfw
    zt_ref[...] = z * alpha[:, None]
    s_ref[...] = z * selfw[:, None] + b_ref[...]


def _dense_mid_body(q0_ref, q1_ref, alpha_ref, selfw_ref, s_ref, w_ref, b_ref,
                    zt_ref, snext_ref):
    alpha = alpha_ref[...]
    h = jnp.tanh(alpha[:, None] * (q0_ref[...] + q1_ref[...]) + s_ref[...])
    z = jnp.dot(h, w_ref[...], preferred_element_type=jnp.float32)
    zt_ref[...] = z * alpha[:, None]
    snext_ref[...] = z * selfw_ref[...][:, None] + b_ref[...]


def _dense_out_body(q0_ref, q1_ref, alpha_ref, s_ref, w_ref, b_ref,
                    out_ref, emb_ref):
    emb = jnp.tanh(alpha_ref[...][:, None] * (q0_ref[...] + q1_ref[...])
                   + s_ref[...])
    emb_ref[...] = emb[:, :emb_ref.shape[1]]
    out_ref[...] = (jnp.dot(emb, w_ref[...], preferred_element_type=jnp.float32)
                    + b_ref[...])


def _row_spec(d):
    return pl.BlockSpec((BN, d), lambda i: (i, 0))


def _vec_spec():
    return pl.BlockSpec((BN,), lambda i: (i,))


def _full_spec(shape):
    return pl.BlockSpec(shape, lambda i: tuple(0 for _ in shape))


def kernel(x, edge_index, W1, b1, W2, b2, W3, b3, Wc, bc):
    n, in_dim = x.shape
    e = edge_index.shape[1]
    hid = W1.shape[1]
    emb_d = W3.shape[1]
    ncls = Wc.shape[1]
    AD = 8
    npad = (n + BN) // BN * BN
    rows_per_tile = math.ceil(e / (NW * IDXW * CB * NIB)) * CB * NIB
    epad = NW * IDXW * rows_per_tile
    grid = (npad // BN,)

    pad_idx = n + jnp.arange(epad - e, dtype=jnp.int32) % (npad - n)
    src2d = jnp.concatenate([edge_index[0], pad_idx]).reshape(epad // IDXW, IDXW)
    dst2d = jnp.concatenate([edge_index[1], pad_idx]).reshape(epad // IDXW, IDXW)
    xp = jnp.zeros((npad, in_dim), jnp.float32).at[:n].set(x)
    ones = jnp.ones((IDXW,), jnp.float32)
    zeros1 = jnp.zeros((npad // NS,), jnp.float32)
    zeros_h = jnp.zeros((npad // NS, AD), jnp.float32)

    def padw(w, b):
        wp = jnp.zeros((w.shape[0], AD), jnp.float32).at[:, :w.shape[1]].set(w)
        bp = jnp.zeros((1, AD), jnp.float32).at[:, :b.shape[0]].set(b)
        return wp, bp

    W1p, b1p = padw(W1, b1)
    W2p, b2p = padw(jnp.zeros((AD, hid), jnp.float32).at[:hid].set(W2), b2)
    W3p, b3p = padw(jnp.zeros((AD, emb_d), jnp.float32).at[:hid].set(W3), b3)
    Wcp = jnp.zeros((AD, ncls), jnp.float32).at[:emb_d].set(Wc)

    degp = _make_deg(npad, epad // IDXW)(dst2d, ones, zeros1)

    dense0 = pl.pallas_call(
        _dense0_body,
        grid=grid,
        in_specs=[pl.BlockSpec((NC, BN), lambda i: (0, i)),
                  _row_spec(in_dim), _full_spec((in_dim, AD)),
                  _full_spec((1, AD))],
        out_specs=(_vec_spec(), _vec_spec(), _row_spec(AD), _row_spec(AD)),
        out_shape=(jax.ShapeDtypeStruct((npad,), jnp.float32),
                   jax.ShapeDtypeStruct((npad,), jnp.float32),
                   jax.ShapeDtypeStruct((npad, AD), jnp.float32),
                   jax.ShapeDtypeStruct((npad, AD), jnp.float32)),
    )
    alpha, selfw, zt1, s1 = dense0(degp, xp, W1p, b1p)

    agg = _make_agg(npad, AD, epad // IDXW)
    q1 = agg(src2d, dst2d, zt1, zeros_h)

    def dense_mid(q, s, wp, bp):
        f = pl.pallas_call(
            _dense_mid_body,
            grid=grid,
            in_specs=[_row_spec(AD), _row_spec(AD), _vec_spec(),
                      _vec_spec(), _row_spec(AD),
                      _full_spec((AD, AD)), _full_spec((1, AD))],
            out_specs=(_row_spec(AD), _row_spec(AD)),
            out_shape=(jax.ShapeDtypeStruct((npad, AD), jnp.float32),
                       jax.ShapeDtypeStruct((npad, AD), jnp.float32)),
        )
        return f(q[0], q[1], alpha, selfw, s, wp, bp)

    zt2, s2 = dense_mid(q1, s1, W2p, b2p)
    q2 = agg(src2d, dst2d, zt2, zeros_h)
    zt3, s3 = dense_mid(q2, s2, W3p, b3p)
    q3 = agg(src2d, dst2d, zt3, zeros_h)

    dense_out = pl.pallas_call(
        _dense_out_body,
        grid=grid,
        in_specs=[_row_spec(AD), _row_spec(AD), _vec_spec(),
                  _row_spec(AD), _full_spec((AD, ncls)),
                  _full_spec((1, ncls))],
        out_specs=(_row_spec(ncls), _row_spec(emb_d)),
        out_shape=(jax.ShapeDtypeStruct((npad, ncls), jnp.float32),
                   jax.ShapeDtypeStruct((npad, emb_d), jnp.float32)),
    )
    out, emb = dense_out(q3[0], q3[1], alpha, s3, Wcp, bc.reshape(1, ncls))
    return out[:n], emb[:n]


# speedup vs baseline: 41.2505x; 1.2898x over previous
import functools
import math

import jax
import jax.numpy as jnp
from jax import lax
from jax.experimental import pallas as pl
from jax.experimental.pallas import tpu as pltpu
from jax.experimental.pallas import tpu_sc as plsc

NC = 2
NS = 16
NW = NC * NS
IDXW = 128
BN = 7168


def _mesh():
    return plsc.VectorSubcoreMesh(
        core_axis_name="c", subcore_axis_name="s",
        num_cores=NC, num_subcores=NS)


NBUF = 4
LOOKAHEAD = 2
CB = 40
NIB = 2

_SC_PARAMS = pltpu.CompilerParams(use_tc_tiling_on_sc=False)


def _make_deg(npad, nrows):
    rpt = nrows // NW
    nblk = rpt // CB

    def body(dst_hbm, ones_hbm, zeros_hbm, deg_out, acc_sh, didx, ones_v,
             *sems):
        ssems = sems[:NBUF]
        isems = sems[NBUF:]
        cid = lax.axis_index("c")
        sid = lax.axis_index("s")
        wid = cid * NS + sid
        slc = npad // NS
        base = wid * rpt

        def iload(g, p):
            pltpu.async_copy(dst_hbm.at[pl.ds(base + g * CB, CB)],
                             didx.at[p], isems[p])

        def iwait(p):
            pltpu.make_async_copy(dst_hbm.at[pl.ds(0, CB)], didx.at[p],
                                  isems[p]).wait()

        def scat(p, j, k):
            pltpu.async_copy(ones_v, acc_sh.at[didx.at[p, j]], ssems[k],
                             add=True)

        def swait(k):
            pltpu.make_async_copy(ones_v, acc_sh.at[didx.at[0, 0]],
                                  ssems[k]).wait()

        pltpu.sync_copy(ones_hbm, ones_v)
        pltpu.sync_copy(zeros_hbm, acc_sh.at[pl.ds(sid * slc, slc)])
        iload(0, 0)
        iwait(0)
        plsc.subcore_barrier()

        def outer(gg, carry):
            for pp in range(NIB):
                g = gg * NIB + pp
                p, pn = pp, (pp + 1) % NIB
                for j in range(CB):
                    r = g * CB + j
                    k = j % NBUF

                    @pl.when(r >= NBUF)
                    def _():
                        swait(k)

                    scat(p, j, k)
                    if j == 8:
                        @pl.when(g + 1 < nblk)
                        def _():
                            iload(g + 1, pn)
                    if j == CB - 2:
                        @pl.when(g + 1 < nblk)
                        def _():
                            iwait(pn)
            return carry

        lax.fori_loop(0, nblk // NIB, outer, 0)
        for k in range(NBUF):
            swait(k)
        plsc.subcore_barrier()
        pltpu.sync_copy(acc_sh.at[pl.ds(sid * slc, slc)],
                        deg_out.at[cid, pl.ds(sid * slc, slc)])

    return pl.kernel(
        body,
        out_type=jax.ShapeDtypeStruct((NC, npad), jnp.float32),
        mesh=_mesh(),
        compiler_params=_SC_PARAMS,
        scratch_types=[
            pltpu.VMEM_SHARED((npad,), jnp.float32),
            pltpu.VMEM((NIB, CB, IDXW), jnp.int32),
            pltpu.VMEM((IDXW,), jnp.float32),
        ] + [pltpu.SemaphoreType.DMA] * (NBUF + NIB),
    )


def _make_agg(npad, d, nrows):
    rpt = nrows // NW
    nblk = rpt // CB

    def body(src_hbm, dst_hbm, zt_hbm, zeros_hbm, q_out,
             acc_sh, sidx, didx, rows, *sems):
        gsems = sems[:NBUF]
        ssems = sems[NBUF:2 * NBUF]
        isems = sems[2 * NBUF:]
        cid = lax.axis_index("c")
        sid = lax.axis_index("s")
        wid = cid * NS + sid
        slc = npad // NS
        base = wid * rpt

        def iload(g, p):
            pltpu.async_copy(src_hbm.at[pl.ds(base + g * CB, CB)],
                             sidx.at[p], isems[p])
            pltpu.async_copy(dst_hbm.at[pl.ds(base + g * CB, CB)],
                             didx.at[p], isems[p])

        def iwait(p):
            pltpu.make_async_copy(src_hbm.at[pl.ds(0, CB)], sidx.at[p],
                                  isems[p]).wait()
            pltpu.make_async_copy(dst_hbm.at[pl.ds(0, CB)], didx.at[p],
                                  isems[p]).wait()

        def gather(p, j, k):
            pltpu.async_copy(zt_hbm.at[sidx.at[p, j]], rows.at[k], gsems[k])

        def gwait(p, j, k):
            pltpu.make_async_copy(zt_hbm.at[sidx.at[p, j]], rows.at[k],
                                  gsems[k]).wait()

        def scat(p, j, k):
            pltpu.async_copy(rows.at[k], acc_sh.at[didx.at[p, j]], ssems[k],
                             add=True)

        def swait(k):
            pltpu.make_async_copy(rows.at[k], acc_sh.at[didx.at[0, 0]],
                                  ssems[k]).wait()

        pltpu.sync_copy(zeros_hbm, acc_sh.at[pl.ds(sid * slc, slc)])
        iload(0, 0)
        iwait(0)
        for j in range(LOOKAHEAD):
            gather(0, j, j)
        plsc.subcore_barrier()

        def outer(gg, carry):
            for pp in range(NIB):
                g = gg * NIB + pp
                p, pn = pp, (pp + 1) % NIB
                for j in range(CB):
                    r = g * CB + j
                    k = j % NBUF
                    nxt = r + LOOKAHEAD
                    jn = j + LOOKAHEAD
                    kn = jn % NBUF

                    @pl.when(jnp.logical_and(nxt >= NBUF, nxt < rpt))
                    def _():
                        swait(kn)

                    if jn < CB:
                        @pl.when(nxt < rpt)
                        def _():
                            gather(p, jn, kn)
                    else:
                        if jn == CB:
                            @pl.when(g + 1 < nblk)
                            def _():
                                iwait(pn)

                        @pl.when(nxt < rpt)
                        def _():
                            gather(pn, jn - CB, kn)

                    gwait(p, j, k)
                    scat(p, j, k)
                    if j == 8:
                        @pl.when(g + 1 < nblk)
                        def _():
                            iload(g + 1, pn)
            return carry

        lax.fori_loop(0, nblk // NIB, outer, 0)
        for k in range(NBUF):
            swait(k)
        plsc.subcore_barrier()
        pltpu.sync_copy(acc_sh.at[pl.ds(sid * slc, slc)],
                        q_out.at[cid, pl.ds(sid * slc, slc)])

    return pl.kernel(
        body,
        out_type=jax.ShapeDtypeStruct((NC, npad, d), jnp.float32),
        mesh=_mesh(),
        compiler_params=_SC_PARAMS,
        scratch_types=[
            pltpu.VMEM_SHARED((npad, d), jnp.float32),
            pltpu.VMEM((NIB, CB, IDXW), jnp.int32),
            pltpu.VMEM((NIB, CB, IDXW), jnp.int32),
            pltpu.VMEM((NBUF, IDXW, d), jnp.float32),
        ] + [pltpu.SemaphoreType.DMA] * (2 * NBUF + NIB),
    )


def _dense0_body(p_ref, x_ref, w_ref, b_ref,
                 alpha_ref, selfw_ref, zt_ref, s_ref):
    deg = p_ref[0, :] + p_ref[1, :] + 1.0
    alpha = lax.rsqrt(deg)
    selfw = 1.0 / deg
    z = jnp.dot(x_ref[...], w_ref[...], preferred_element_type=jnp.float32)
    alpha_ref[...] = alpha
    selfw_ref[...] = selfw
    zt_ref[...] = z * alpha[:, None]
    s_ref[...] = z * selfw[:, None] + b_ref[...]


def _dense_mid_body(q_ref, alpha_ref, selfw_ref, s_ref, w_ref, b_ref,
                    zt_ref, snext_ref):
    alpha = alpha_ref[...]
    h = jnp.tanh(alpha[:, None] * (q_ref[0] + q_ref[1]) + s_ref[...])
    z = jnp.dot(h, w_ref[...], preferred_element_type=jnp.float32)
    zt_ref[...] = z * alpha[:, None]
    snext_ref[...] = z * selfw_ref[...][:, None] + b_ref[...]


def _dense_out_body(q_ref, alpha_ref, s_ref, w_ref, b_ref,
                    out_ref, emb_ref):
    emb = jnp.tanh(alpha_ref[...][:, None] * (q_ref[0] + q_ref[1])
                   + s_ref[...])
    emb_ref[...] = emb[:, :emb_ref.shape[1]]
    out_ref[...] = (jnp.dot(emb, w_ref[...], preferred_element_type=jnp.float32)
                    + b_ref[...])


def _row_spec(d):
    return pl.BlockSpec((BN, d), lambda i: (i, 0))


def _vec_spec():
    return pl.BlockSpec((BN,), lambda i: (i,))


def _full_spec(shape):
    return pl.BlockSpec(shape, lambda i: tuple(0 for _ in shape))


def kernel(x, edge_index, W1, b1, W2, b2, W3, b3, Wc, bc):
    n, in_dim = x.shape
    e = edge_index.shape[1]
    hid = W1.shape[1]
    emb_d = W3.shape[1]
    ncls = Wc.shape[1]
    AD = 8
    npad = (n + BN) // BN * BN
    rows_per_tile = math.ceil(e / (NW * IDXW * CB * NIB)) * CB * NIB
    epad = NW * IDXW * rows_per_tile
    grid = (npad // BN,)

    pad_idx = n + jnp.arange(epad - e, dtype=jnp.int32) % (npad - n)
    src2d = jnp.concatenate([edge_index[0], pad_idx]).reshape(epad // IDXW, IDXW)
    dst2d = jnp.concatenate([edge_index[1], pad_idx]).reshape(epad // IDXW, IDXW)
    xp = jnp.zeros((npad, in_dim), jnp.float32).at[:n].set(x)
    ones = jnp.ones((IDXW,), jnp.float32)
    zeros1 = jnp.zeros((npad // NS,), jnp.float32)
    zeros_h = jnp.zeros((npad // NS, AD), jnp.float32)

    def padw(w, b):
        wp = jnp.zeros((w.shape[0], AD), jnp.float32).at[:, :w.shape[1]].set(w)
        bp = jnp.zeros((1, AD), jnp.float32).at[:, :b.shape[0]].set(b)
        return wp, bp

    W1p, b1p = padw(W1, b1)
    W2p, b2p = padw(jnp.zeros((AD, hid), jnp.float32).at[:hid].set(W2), b2)
    W3p, b3p = padw(jnp.zeros((AD, emb_d), jnp.float32).at[:hid].set(W3), b3)
    Wcp = jnp.zeros((AD, ncls), jnp.float32).at[:emb_d].set(Wc)

    degp = _make_deg(npad, epad // IDXW)(dst2d, ones, zeros1)

    dense0 = pl.pallas_call(
        _dense0_body,
        grid=grid,
        in_specs=[pl.BlockSpec((NC, BN), lambda i: (0, i)),
                  _row_spec(in_dim), _full_spec((in_dim, AD)),
                  _full_spec((1, AD))],
        out_specs=(_vec_spec(), _vec_spec(), _row_spec(AD), _row_spec(AD)),
        out_shape=(jax.ShapeDtypeStruct((npad,), jnp.float32),
                   jax.ShapeDtypeStruct((npad,), jnp.float32),
                   jax.ShapeDtypeStruct((npad, AD), jnp.float32),
                   jax.ShapeDtypeStruct((npad, AD), jnp.float32)),
    )
    alpha, selfw, zt1, s1 = dense0(degp, xp, W1p, b1p)

    agg = _make_agg(npad, AD, epad // IDXW)
    q1 = agg(src2d, dst2d, zt1, zeros_h)

    def dense_mid(q, s, wp, bp):
        f = pl.pallas_call(
            _dense_mid_body,
            grid=grid,
            in_specs=[pl.BlockSpec((NC, BN, AD), lambda i: (0, i, 0)),
                      _vec_spec(), _vec_spec(), _row_spec(AD),
                      _full_spec((AD, AD)), _full_spec((1, AD))],
            out_specs=(_row_spec(AD), _row_spec(AD)),
            out_shape=(jax.ShapeDtypeStruct((npad, AD), jnp.float32),
                       jax.ShapeDtypeStruct((npad, AD), jnp.float32)),
        )
        return f(q, alpha, selfw, s, wp, bp)

    zt2, s2 = dense_mid(q1, s1, W2p, b2p)
    q2 = agg(src2d, dst2d, zt2, zeros_h)
    zt3, s3 = dense_mid(q2, s2, W3p, b3p)
    q3 = agg(src2d, dst2d, zt3, zeros_h)

    dense_out = pl.pallas_call(
        _dense_out_body,
        grid=grid,
        in_specs=[pl.BlockSpec((NC, BN, AD), lambda i: (0, i, 0)),
                  _vec_spec(), _row_spec(AD), _full_spec((AD, ncls)),
                  _full_spec((1, ncls))],
        out_specs=(_row_spec(ncls), _row_spec(emb_d)),
        out_shape=(jax.ShapeDtypeStruct((npad, ncls), jnp.float32),
                   jax.ShapeDtypeStruct((npad, emb_d), jnp.float32)),
    )
    out, emb = dense_out(q3, alpha, s3, Wcp, bc.reshape(1, ncls))
    return out[:n], emb[:n]


# speedup vs baseline: 63.5076x; 1.5396x over previous
import functools
import math

import jax
import jax.numpy as jnp
from jax import lax
from jax.experimental import pallas as pl
from jax.experimental.pallas import tpu as pltpu
from jax.experimental.pallas import tpu_sc as plsc

NC = 2
NS = 16
NW = NC * NS
IDXW = 128
PK = 16
BN = 14336


def _mesh():
    return plsc.VectorSubcoreMesh(
        core_axis_name="c", subcore_axis_name="s",
        num_cores=NC, num_subcores=NS)


NBUF = 4
LOOKAHEAD = 2
CB = 40
NIB = 2

_SC_PARAMS = pltpu.CompilerParams(use_tc_tiling_on_sc=False)


def _make_deg(npad, d, nrows):
    rpt = nrows // NW
    nblk = rpt // CB

    def body(dst_hbm, ones_hbm, zeros_hbm, deg_out, acc_sh, didx, ones_v,
             *sems):
        ssems = sems[:NBUF]
        isems = sems[NBUF:]
        cid = lax.axis_index("c")
        sid = lax.axis_index("s")
        wid = cid * NS + sid
        slc = npad // NS
        base = wid * rpt

        def iload(g, p):
            pltpu.async_copy(dst_hbm.at[pl.ds(base + g * CB, CB)],
                             didx.at[p], isems[p])

        def iwait(p):
            pltpu.make_async_copy(dst_hbm.at[pl.ds(0, CB)], didx.at[p],
                                  isems[p]).wait()

        def scat(p, j, k):
            pltpu.async_copy(ones_v, acc_sh.at[didx.at[p, j]], ssems[k],
                             add=True)

        def swait(k):
            pltpu.make_async_copy(ones_v, acc_sh.at[didx.at[0, 0]],
                                  ssems[k]).wait()

        pltpu.sync_copy(ones_hbm, ones_v)
        pltpu.sync_copy(zeros_hbm, acc_sh.at[pl.ds(sid * slc, slc)])
        iload(0, 0)
        iwait(0)
        plsc.subcore_barrier()

        def outer(gg, carry):
            for pp in range(NIB):
                g = gg * NIB + pp
                p, pn = pp, (pp + 1) % NIB
                for j in range(CB):
                    r = g * CB + j
                    k = j % NBUF

                    @pl.when(r >= NBUF)
                    def _():
                        swait(k)

                    scat(p, j, k)
                    if j == 8:
                        @pl.when(g + 1 < nblk)
                        def _():
                            iload(g + 1, pn)
                    if j == CB - 2:
                        @pl.when(g + 1 < nblk)
                        def _():
                            iwait(pn)
            return carry

        lax.fori_loop(0, nblk // NIB, outer, 0)
        for k in range(NBUF):
            swait(k)
        plsc.subcore_barrier()
        pltpu.sync_copy(acc_sh.at[pl.ds(sid * slc, slc)],
                        deg_out.at[cid, pl.ds(sid * slc, slc)])

    return pl.kernel(
        body,
        out_type=jax.ShapeDtypeStruct((NC, npad, d), jnp.float32),
        mesh=_mesh(),
        compiler_params=_SC_PARAMS,
        scratch_types=[
            pltpu.VMEM_SHARED((npad, d), jnp.float32),
            pltpu.VMEM((NIB, CB, IDXW), jnp.int32),
            pltpu.VMEM((IDXW, d), jnp.float32),
        ] + [pltpu.SemaphoreType.DMA] * (NBUF + NIB),
    )


def _make_agg(npad, d, nrows):
    rpt = nrows // NW
    nblk = rpt // CB

    def body(src_hbm, dst_hbm, zt_hbm, zeros_hbm, q_out,
             acc_sh, sidx, didx, rows, *sems):
        gsems = sems[:NBUF]
        ssems = sems[NBUF:2 * NBUF]
        isems = sems[2 * NBUF:]
        cid = lax.axis_index("c")
        sid = lax.axis_index("s")
        wid = cid * NS + sid
        slc = npad // NS
        base = wid * rpt

        def iload(g, p):
            pltpu.async_copy(src_hbm.at[pl.ds(base + g * CB, CB)],
                             sidx.at[p], isems[p])
            pltpu.async_copy(dst_hbm.at[pl.ds(base + g * CB, CB)],
                             didx.at[p], isems[p])

        def iwait(p):
            pltpu.make_async_copy(src_hbm.at[pl.ds(0, CB)], sidx.at[p],
                                  isems[p]).wait()
            pltpu.make_async_copy(dst_hbm.at[pl.ds(0, CB)], didx.at[p],
                                  isems[p]).wait()

        def gather(p, j, k):
            pltpu.async_copy(zt_hbm.at[sidx.at[p, j]], rows.at[k], gsems[k])

        def gwait(p, j, k):
            pltpu.make_async_copy(zt_hbm.at[sidx.at[p, j]], rows.at[k],
                                  gsems[k]).wait()

        def scat(p, j, k):
            pltpu.async_copy(rows.at[k], acc_sh.at[didx.at[p, j]], ssems[k],
                             add=True)

        def swait(k):
            pltpu.make_async_copy(rows.at[k], acc_sh.at[didx.at[0, 0]],
                                  ssems[k]).wait()

        pltpu.sync_copy(zeros_hbm, acc_sh.at[pl.ds(sid * slc, slc)])
        iload(0, 0)
        iwait(0)
        for j in range(LOOKAHEAD):
            gather(0, j, j)
        plsc.subcore_barrier()

        def outer(gg, carry):
            for pp in range(NIB):
                g = gg * NIB + pp
                p, pn = pp, (pp + 1) % NIB
                for j in range(CB):
                    r = g * CB + j
                    k = j % NBUF
                    nxt = r + LOOKAHEAD
                    jn = j + LOOKAHEAD
                    kn = jn % NBUF

                    @pl.when(jnp.logical_and(nxt >= NBUF, nxt < rpt))
                    def _():
                        swait(kn)

                    if jn < CB:
                        @pl.when(nxt < rpt)
                        def _():
                            gather(p, jn, kn)
                    else:
                        if jn == CB:
                            @pl.when(g + 1 < nblk)
                            def _():
                                iwait(pn)

                        @pl.when(nxt < rpt)
                        def _():
                            gather(pn, jn - CB, kn)

                    gwait(p, j, k)
                    scat(p, j, k)
                    if j == 8:
                        @pl.when(g + 1 < nblk)
                        def _():
                            iload(g + 1, pn)
            return carry

        lax.fori_loop(0, nblk // NIB, outer, 0)
        for k in range(NBUF):
            swait(k)
        plsc.subcore_barrier()
        pltpu.sync_copy(acc_sh.at[pl.ds(sid * slc, slc)],
                        q_out.at[cid, pl.ds(sid * slc, slc)])

    return pl.kernel(
        body,
        out_type=jax.ShapeDtypeStruct((NC, npad, d), jnp.float32),
        mesh=_mesh(),
        compiler_params=_SC_PARAMS,
        scratch_types=[
            pltpu.VMEM_SHARED((npad, d), jnp.float32),
            pltpu.VMEM((NIB, CB, IDXW), jnp.int32),
            pltpu.VMEM((NIB, CB, IDXW), jnp.int32),
            pltpu.VMEM((NBUF, IDXW, d), jnp.float32),
        ] + [pltpu.SemaphoreType.DMA] * (2 * NBUF + NIB),
    )


def _matmul0_body(x_ref, w_ref, z_ref):
    z_ref[...] = jnp.dot(x_ref[...], w_ref[...],
                         preferred_element_type=jnp.float32)


def _dense0_body(p_ref, z_ref, brep_ref,
                 alpha_ref, selfw_ref, zt_ref, s_ref):
    deg = p_ref[0] + p_ref[1] + 1.0
    alpha = lax.rsqrt(deg)
    selfw = 1.0 / deg
    z = z_ref[...]
    alpha_ref[...] = alpha
    selfw_ref[...] = selfw
    zt_ref[...] = z * alpha
    s_ref[...] = z * selfw + brep_ref[...]


def _dense_mid_body(q_ref, alpha_ref, selfw_ref, s_ref, bd_ref, brep_ref,
                    zt_ref, snext_ref):
    alpha = alpha_ref[...]
    h = jnp.tanh(alpha * (q_ref[0] + q_ref[1]) + s_ref[...])
    z = jnp.dot(h, bd_ref[...], preferred_element_type=jnp.float32)
    zt_ref[...] = z * alpha
    snext_ref[...] = z * selfw_ref[...] + brep_ref[...]


def _dense_out_body(q_ref, alpha_ref, s_ref, bd_ref, brep_ref,
                    out_ref, emb_ref):
    emb = jnp.tanh(alpha_ref[...] * (q_ref[0] + q_ref[1]) + s_ref[...])
    emb_ref[...] = emb
    out_ref[...] = (jnp.dot(emb, bd_ref[...],
                            preferred_element_type=jnp.float32)
                    + brep_ref[...])


def _pk_spec():
    return pl.BlockSpec((BN // PK, 128), lambda i: (i, 0))


def _qpk_spec():
    return pl.BlockSpec((NC, BN // PK, 128), lambda i: (0, i, 0))


def _full_spec(shape):
    return pl.BlockSpec(shape, lambda i: tuple(0 for _ in shape))


def kernel(x, edge_index, W1, b1, W2, b2, W3, b3, Wc, bc):
    n, in_dim = x.shape
    e = edge_index.shape[1]
    hid = W1.shape[1]
    emb_d = W3.shape[1]
    ncls = Wc.shape[1]
    AD = 8
    npad = (n + BN) // BN * BN
    nrows = npad // PK
    bnp = BN // PK
    rows_per_tile = math.ceil(e / (NW * IDXW * CB * NIB)) * CB * NIB
    epad = NW * IDXW * rows_per_tile
    grid = (npad // BN,)

    pad_idx = n + jnp.arange(epad - e, dtype=jnp.int32) % (npad - n)
    src2d = jnp.concatenate([edge_index[0], pad_idx]).reshape(epad // IDXW, IDXW)
    dst2d = jnp.concatenate([edge_index[1], pad_idx]).reshape(epad // IDXW, IDXW)
    ones = jnp.ones((IDXW, AD), jnp.float32)
    zeros_h = jnp.zeros((npad // NS, AD), jnp.float32)

    def padw(w):
        return jnp.zeros((AD, AD), jnp.float32).at[:w.shape[0],
                                                   :w.shape[1]].set(w)

    def brep(b):
        bp = jnp.zeros((AD,), jnp.float32).at[:b.shape[0]].set(b)
        return jnp.tile(bp, PK).reshape(1, PK * AD)

    eye = jnp.eye(PK, dtype=jnp.float32)
    W1p = jnp.zeros((in_dim, AD), jnp.float32).at[:, :hid].set(W1)
    BD2 = jnp.kron(eye, padw(W2))
    BD3 = jnp.kron(eye, padw(W3))
    BDc = jnp.kron(eye, padw(Wc))

    degp = _make_deg(npad, AD, epad // IDXW)(dst2d, ones, zeros_h)
    degp_v = degp.reshape(NC, nrows, PK * AD)

    matmul0 = pl.pallas_call(
        _matmul0_body,
        grid=grid,
        in_specs=[pl.BlockSpec((BN, in_dim), lambda i: (i, 0)),
                  _full_spec((in_dim, AD))],
        out_specs=pl.BlockSpec((BN, AD), lambda i: (i, 0)),
        out_shape=jax.ShapeDtypeStruct((npad, AD), jnp.float32),
    )
    z1_pk = matmul0(x, W1p).reshape(nrows, PK * AD)

    dense0 = pl.pallas_call(
        _dense0_body,
        grid=grid,
        in_specs=[_qpk_spec(), _pk_spec(), _full_spec((1, PK * AD))],
        out_specs=(_pk_spec(), _pk_spec(), _pk_spec(), _pk_spec()),
        out_shape=tuple(jax.ShapeDtypeStruct((nrows, PK * AD), jnp.float32)
                        for _ in range(4)),
    )
    alpha, selfw, zt1, s1 = dense0(degp_v, z1_pk, brep(b1))

    agg = _make_agg(npad, AD, epad // IDXW)

    def dense_mid(q, s, bd, br):
        f = pl.pallas_call(
            _dense_mid_body,
            grid=grid,
            in_specs=[_qpk_spec(), _pk_spec(), _pk_spec(), _pk_spec(),
                      _full_spec((PK * AD, PK * AD)), _full_spec((1, PK * AD))],
            out_specs=(_pk_spec(), _pk_spec()),
            out_shape=(jax.ShapeDtypeStruct((nrows, PK * AD), jnp.float32),
                       jax.ShapeDtypeStruct((nrows, PK * AD), jnp.float32)),
        )
        return f(q.reshape(NC, nrows, PK * AD), alpha, selfw, s, bd, br)

    q1 = agg(src2d, dst2d, zt1.reshape(npad, AD), zeros_h)
    zt2, s2 = dense_mid(q1, s1, BD2, brep(b2))
    q2 = agg(src2d, dst2d, zt2.reshape(npad, AD), zeros_h)
    zt3, s3 = dense_mid(q2, s2, BD3, brep(b3))
    q3 = agg(src2d, dst2d, zt3.reshape(npad, AD), zeros_h)

    dense_out = pl.pallas_call(
        _dense_out_body,
        grid=grid,
        in_specs=[_qpk_spec(), _pk_spec(), _pk_spec(),
                  _full_spec((PK * AD, PK * AD)), _full_spec((1, PK * AD))],
        out_specs=(_pk_spec(), _pk_spec()),
        out_shape=(jax.ShapeDtypeStruct((nrows, PK * AD), jnp.float32),
                   jax.ShapeDtypeStruct((nrows, PK * AD), jnp.float32)),
    )
    out_pk, emb_pk = dense_out(q3.reshape(NC, nrows, PK * AD), alpha, s3,
                               BDc, brep(bc))
    out = out_pk.reshape(npad, AD)[:n, :ncls]
    emb = emb_pk.reshape(npad, AD)[:n, :emb_d]
    return out, emb


# speedup vs baseline: 72.9198x; 1.1482x over previous
import functools
import math

import jax
import jax.numpy as jnp
from jax import lax
from jax.experimental import pallas as pl
from jax.experimental.pallas import tpu as pltpu
from jax.experimental.pallas import tpu_sc as plsc

NC = 2
NS = 16
NW = NC * NS
IDXW = 1024
PK = 16
BN = 14336


def _mesh():
    return plsc.VectorSubcoreMesh(
        core_axis_name="c", subcore_axis_name="s",
        num_cores=NC, num_subcores=NS)


NBUF = 2
LOOKAHEAD = 1
CB = 2
NIB = 2

_SC_PARAMS = pltpu.CompilerParams(use_tc_tiling_on_sc=False)


def _make_deg(npad, d, nrows):
    rpt = nrows // NW
    nblk = rpt // CB

    def body(dst_hbm, ones_hbm, zeros_hbm, deg_out, acc_sh, didx, ones_v,
             *sems):
        ssems = sems[:NBUF]
        isems = sems[NBUF:]
        cid = lax.axis_index("c")
        sid = lax.axis_index("s")
        wid = cid * NS + sid
        slc = npad // NS
        base = wid * rpt

        def iload(g, p):
            pltpu.async_copy(dst_hbm.at[pl.ds(base + g * CB, CB)],
                             didx.at[p], isems[p])

        def iwait(p):
            pltpu.make_async_copy(dst_hbm.at[pl.ds(0, CB)], didx.at[p],
                                  isems[p]).wait()

        def scat(p, j, k):
            pltpu.async_copy(ones_v, acc_sh.at[didx.at[p, j]], ssems[k],
                             add=True)

        def swait(k):
            pltpu.make_async_copy(ones_v, acc_sh.at[didx.at[0, 0]],
                                  ssems[k]).wait()

        pltpu.sync_copy(ones_hbm, ones_v)
        pltpu.sync_copy(zeros_hbm, acc_sh.at[pl.ds(sid * slc, slc)])
        iload(0, 0)
        iwait(0)
        plsc.subcore_barrier()

        def outer(gg, carry):
            for pp in range(NIB):
                g = gg * NIB + pp
                p, pn = pp, (pp + 1) % NIB
                for j in range(CB):
                    r = g * CB + j
                    k = j % NBUF

                    @pl.when(r >= NBUF)
                    def _():
                        swait(k)

                    scat(p, j, k)
                    if j == CB - 1:
                        @pl.when(g + 1 < nblk)
                        def _():
                            iload(g + 1, pn)
                            iwait(pn)
            return carry

        lax.fori_loop(0, nblk // NIB, outer, 0)
        for k in range(NBUF):
            swait(k)
        plsc.subcore_barrier()
        pltpu.sync_copy(acc_sh.at[pl.ds(sid * slc, slc)],
                        deg_out.at[cid, pl.ds(sid * slc, slc)])

    return pl.kernel(
        body,
        out_type=jax.ShapeDtypeStruct((NC, npad, d), jnp.float32),
        mesh=_mesh(),
        compiler_params=_SC_PARAMS,
        scratch_types=[
            pltpu.VMEM_SHARED((npad, d), jnp.float32),
            pltpu.VMEM((NIB, CB, IDXW), jnp.int32),
            pltpu.VMEM((IDXW, d), jnp.float32),
        ] + [pltpu.SemaphoreType.DMA] * (NBUF + NIB),
    )


def _make_agg(npad, d, nrows):
    rpt = nrows // NW
    nblk = rpt // CB

    def body(src_hbm, dst_hbm, zt_hbm, zeros_hbm, q_out,
             acc_sh, sidx, didx, rows, *sems):
        gsems = sems[:NBUF]
        ssems = sems[NBUF:2 * NBUF]
        isems = sems[2 * NBUF:]
        cid = lax.axis_index("c")
        sid = lax.axis_index("s")
        wid = cid * NS + sid
        slc = npad // NS
        base = wid * rpt

        def iload(g, p):
            pltpu.async_copy(src_hbm.at[pl.ds(base + g * CB, CB)],
                             sidx.at[p], isems[p])
            pltpu.async_copy(dst_hbm.at[pl.ds(base + g * CB, CB)],
                             didx.at[p], isems[p])

        def iwait(p):
            pltpu.make_async_copy(src_hbm.at[pl.ds(0, CB)], sidx.at[p],
                                  isems[p]).wait()
            pltpu.make_async_copy(dst_hbm.at[pl.ds(0, CB)], didx.at[p],
                                  isems[p]).wait()

        def gather(p, j, k):
            pltpu.async_copy(zt_hbm.at[sidx.at[p, j]], rows.at[k], gsems[k])

        def gwait(p, j, k):
            pltpu.make_async_copy(zt_hbm.at[sidx.at[p, j]], rows.at[k],
                                  gsems[k]).wait()

        def scat(p, j, k):
            pltpu.async_copy(rows.at[k], acc_sh.at[didx.at[p, j]], ssems[k],
                             add=True)

        def swait(k):
            pltpu.make_async_copy(rows.at[k], acc_sh.at[didx.at[0, 0]],
                                  ssems[k]).wait()

        pltpu.sync_copy(zeros_hbm, acc_sh.at[pl.ds(sid * slc, slc)])
        iload(0, 0)
        iwait(0)
        for j in range(LOOKAHEAD):
            gather(0, j, j)
        plsc.subcore_barrier()

        def outer(gg, carry):
            for pp in range(NIB):
                g = gg * NIB + pp
                p, pn = pp, (pp + 1) % NIB
                for j in range(CB):
                    r = g * CB + j
                    k = j % NBUF
                    nxt = r + LOOKAHEAD
                    jn = j + LOOKAHEAD
                    kn = jn % NBUF

                    @pl.when(jnp.logical_and(nxt >= NBUF, nxt < rpt))
                    def _():
                        swait(kn)

                    if jn < CB:
                        @pl.when(nxt < rpt)
                        def _():
                            gather(p, jn, kn)
                    else:
                        if jn == CB:
                            @pl.when(g + 1 < nblk)
                            def _():
                                iwait(pn)

                        @pl.when(nxt < rpt)
                        def _():
                            gather(pn, jn - CB, kn)

                    gwait(p, j, k)
                    scat(p, j, k)
                    if j == 0:
                        @pl.when(g + 1 < nblk)
                        def _():
                            iload(g + 1, pn)
            return carry

        lax.fori_loop(0, nblk // NIB, outer, 0)
        for k in range(NBUF):
            swait(k)
        plsc.subcore_barrier()
        pltpu.sync_copy(acc_sh.at[pl.ds(sid * slc, slc)],
                        q_out.at[cid, pl.ds(sid * slc, slc)])

    return pl.kernel(
        body,
        out_type=jax.ShapeDtypeStruct((NC, npad, d), jnp.float32),
        mesh=_mesh(),
        compiler_params=_SC_PARAMS,
        scratch_types=[
            pltpu.VMEM_SHARED((npad, d), jnp.float32),
            pltpu.VMEM((NIB, CB, IDXW), jnp.int32),
            pltpu.VMEM((NIB, CB, IDXW), jnp.int32),
            pltpu.VMEM((NBUF, IDXW, d), jnp.float32),
        ] + [pltpu.SemaphoreType.DMA] * (2 * NBUF + NIB),
    )


def _matmul0_body(x_ref, w_ref, z_ref):
    z_ref[...] = jnp.dot(x_ref[...], w_ref[...],
                         preferred_element_type=jnp.float32)


def _dense0_body(p_ref, z_ref, brep_ref,
                 alpha_ref, selfw_ref, zt_ref, s_ref):
    deg = p_ref[0] + p_ref[1] + 1.0
    alpha = lax.rsqrt(deg)
    selfw = 1.0 / deg
    z = z_ref[...]
    alpha_ref[...] = alpha
    selfw_ref[...] = selfw
    zt_ref[...] = z * alpha
    s_ref[...] = z * selfw + brep_ref[...]


def _dense_mid_body(q_ref, alpha_ref, selfw_ref, s_ref, bd_ref, brep_ref,
                    zt_ref, snext_ref):
    alpha = alpha_ref[...]
    h = jnp.tanh(alpha * (q_ref[0] + q_ref[1]) + s_ref[...])
    z = jnp.dot(h, bd_ref[...], preferred_element_type=jnp.float32)
    zt_ref[...] = z * alpha
    snext_ref[...] = z * selfw_ref[...] + brep_ref[...]


def _dense_out_body(q_ref, alpha_ref, s_ref, bd_ref, brep_ref,
                    out_ref, emb_ref):
    emb = jnp.tanh(alpha_ref[...] * (q_ref[0] + q_ref[1]) + s_ref[...])
    emb_ref[...] = emb
    out_ref[...] = (jnp.dot(emb, bd_ref[...],
                            preferred_element_type=jnp.float32)
                    + brep_ref[...])


def _pk_spec():
    return pl.BlockSpec((BN // PK, 128), lambda i: (i, 0))


def _qpk_spec():
    return pl.BlockSpec((NC, BN // PK, 128), lambda i: (0, i, 0))


def _full_spec(shape):
    return pl.BlockSpec(shape, lambda i: tuple(0 for _ in shape))


def kernel(x, edge_index, W1, b1, W2, b2, W3, b3, Wc, bc):
    n, in_dim = x.shape
    e = edge_index.shape[1]
    hid = W1.shape[1]
    emb_d = W3.shape[1]
    ncls = Wc.shape[1]
    AD = 8
    npad = (n + BN) // BN * BN
    nrows = npad // PK
    bnp = BN // PK
    rows_per_tile = math.ceil(e / (NW * IDXW * CB * NIB)) * CB * NIB
    epad = NW * IDXW * rows_per_tile
    grid = (npad // BN,)

    pad_idx = n + jnp.arange(epad - e, dtype=jnp.int32) % (npad - n)
    src2d = jnp.concatenate([edge_index[0], pad_idx]).reshape(epad // IDXW, IDXW)
    dst2d = jnp.concatenate([edge_index[1], pad_idx]).reshape(epad // IDXW, IDXW)
    ones = jnp.ones((IDXW, AD), jnp.float32)
    zeros_h = jnp.zeros((npad // NS, AD), jnp.float32)

    def padw(w):
        return jnp.zeros((AD, AD), jnp.float32).at[:w.shape[0],
                                                   :w.shape[1]].set(w)

    def brep(b):
        bp = jnp.zeros((AD,), jnp.float32).at[:b.shape[0]].set(b)
        return jnp.tile(bp, PK).reshape(1, PK * AD)

    eye = jnp.eye(PK, dtype=jnp.float32)
    W1p = jnp.zeros((in_dim, AD), jnp.float32).at[:, :hid].set(W1)
    BD2 = jnp.kron(eye, padw(W2))
    BD3 = jnp.kron(eye, padw(W3))
    BDc = jnp.kron(eye, padw(Wc))

    degp = _make_deg(npad, AD, epad // IDXW)(dst2d, ones, zeros_h)
    degp_v = degp.reshape(NC, nrows, PK * AD)

    matmul0 = pl.pallas_call(
        _matmul0_body,
        grid=grid,
        in_specs=[pl.BlockSpec((BN, in_dim), lambda i: (i, 0)),
                  _full_spec((in_dim, AD))],
        out_specs=pl.BlockSpec((BN, AD), lambda i: (i, 0)),
        out_shape=jax.ShapeDtypeStruct((npad, AD), jnp.float32),
    )
    z1_pk = matmul0(x, W1p).reshape(nrows, PK * AD)

    dense0 = pl.pallas_call(
        _dense0_body,
        grid=grid,
        in_specs=[_qpk_spec(), _pk_spec(), _full_spec((1, PK * AD))],
        out_specs=(_pk_spec(), _pk_spec(), _pk_spec(), _pk_spec()),
        out_shape=tuple(jax.ShapeDtypeStruct((nrows, PK * AD), jnp.float32)
                        for _ in range(4)),
    )
    alpha, selfw, zt1, s1 = dense0(degp_v, z1_pk, brep(b1))

    agg = _make_agg(npad, AD, epad // IDXW)

    def dense_mid(q, s, bd, br):
        f = pl.pallas_call(
            _dense_mid_body,
            grid=grid,
            in_specs=[_qpk_spec(), _pk_spec(), _pk_spec(), _pk_spec(),
                      _full_spec((PK * AD, PK * AD)), _full_spec((1, PK * AD))],
            out_specs=(_pk_spec(), _pk_spec()),
            out_shape=(jax.ShapeDtypeStruct((nrows, PK * AD), jnp.float32),
                       jax.ShapeDtypeStruct((nrows, PK * AD), jnp.float32)),
        )
        return f(q.reshape(NC, nrows, PK * AD), alpha, selfw, s, bd, br)

    q1 = agg(src2d, dst2d, zt1.reshape(npad, AD), zeros_h)
    zt2, s2 = dense_mid(q1, s1, BD2, brep(b2))
    q2 = agg(src2d, dst2d, zt2.reshape(npad, AD), zeros_h)
    zt3, s3 = dense_mid(q2, s2, BD3, brep(b3))
    q3 = agg(src2d, dst2d, zt3.reshape(npad, AD), zeros_h)

    dense_out = pl.pallas_call(
        _dense_out_body,
        grid=grid,
        in_specs=[_qpk_spec(), _pk_spec(), _pk_spec(),
                  _full_spec((PK * AD, PK * AD)), _full_spec((1, PK * AD))],
        out_specs=(_pk_spec(), _pk_spec()),
        out_shape=(jax.ShapeDtypeStruct((nrows, PK * AD), jnp.float32),
                   jax.ShapeDtypeStruct((nrows, PK * AD), jnp.float32)),
    )
    out_pk, emb_pk = dense_out(q3.reshape(NC, nrows, PK * AD), alpha, s3,
                               BDc, brep(bc))
    out = out_pk.reshape(npad, AD)[:n, :ncls]
    emb = emb_pk.reshape(npad, AD)[:n, :emb_d]
    return out, emb
